# baseline (speedup 1.0000x reference)
import numpy as np
import jax
import jax.numpy as jnp
from jax import lax
from jax.experimental import pallas as pl
from jax.experimental.pallas import tpu as pltpu
from jax.experimental.pallas import tpu_sc as plsc

VOCAB = 8192
D_MODEL = 768
BATCH = 4

NUM_CORES = 2
NUM_SUBCORES = 16
LANES = 16
NW = NUM_CORES * NUM_SUBCORES
POS_PER_W = VOCAB // NW
CHUNK = 8
NCHUNK = POS_PER_W // CHUNK
NBUF = 4
ROWS = BATCH * CHUNK
NSLICE = D_MODEL // LANES


def _make_pe() -> np.ndarray:
    pos = np.arange(VOCAB, dtype=np.float32)[:, None]
    i = np.arange(D_MODEL, dtype=np.float32)[None, :]
    denom = np.power(np.float32(10000.0),
                     (np.float32(2.0) * i) / np.float32(D_MODEL))
    angle = pos / denom
    even = (np.arange(D_MODEL)[None, :] % 2) == 0
    return np.where(even, np.sin(angle), np.cos(angle)).astype(np.float32)


_PE = _make_pe()


def _body(xt_hbm, table_hbm, pe_hbm, out_hbm,
          idx_f, dix, pe_v, rows_v, gsem, ssem):
    w = lax.axis_index("s") * NUM_CORES + lax.axis_index("c")
    base = w * POS_PER_W

    pltpu.sync_copy(xt_hbm.at[pl.ds(w * NCHUNK * ROWS, NCHUNK * ROWS)], idx_f)

    lane = lax.iota(jnp.int32, LANES)
    dst0 = base + (lane >> 3) * VOCAB + (lane & (CHUNK - 1))

    def stage_body(c, carry):
        off = c * CHUNK
        dix[c, pl.ds(0, LANES)] = dst0 + off
        dix[c, pl.ds(LANES, LANES)] = dst0 + off + 2 * VOCAB
        return carry

    lax.fori_loop(0, NCHUNK, stage_body, 0)

    def fire_gathers(c, s):
        pos = base + c * CHUNK
        pltpu.async_copy(pe_hbm.at[pl.ds(pos, CHUNK)], pe_v.at[s], gsem.at[s])
        pltpu.async_copy(table_hbm.at[idx_f.at[pl.ds(c * ROWS, ROWS)]],
                         rows_v.at[s], gsem.at[s])

    def wait_gathers(c, s):
        pos = base + c * CHUNK
        pltpu.make_async_copy(
            pe_hbm.at[pl.ds(pos, CHUNK)], pe_v.at[s], gsem.at[s]).wait()
        pltpu.make_async_copy(
            table_hbm.at[idx_f.at[pl.ds(c * ROWS, ROWS)]],
            rows_v.at[s], gsem.at[s]).wait()

    def fire_scatter(c, s):
        pltpu.async_copy(rows_v.at[s], out_hbm.at[dix.at[c]], ssem.at[s])

    def wait_scatter(c, s):
        pltpu.make_async_copy(
            rows_v.at[s], out_hbm.at[dix.at[c]], ssem.at[s]).wait()

    def add_chunk(s, unroll):
        def jbody(j, carry):
            def kbody(k, carry2):
                for u in range(unroll):
                    sl = pl.ds(k * (unroll * LANES) + u * LANES, LANES)
                    p = pe_v[s, j, sl]
                    for b in range(BATCH):
                        rows_v[s, b * CHUNK + j, sl] = \
                            rows_v[s, b * CHUNK + j, sl] + p
                return carry2
            return lax.fori_loop(0, NSLICE // unroll, kbody, carry)
        lax.fori_loop(0, CHUNK, jbody, 0)

    def run_chunk(c, s, head, tail, unroll):
        wait_gathers(c, s)
        add_chunk(s, unroll)
        fire_scatter(c, s)
        if not head:
            wait_scatter(c - 2, (s + 2) % NBUF)
        if not tail:
            fire_gathers(c + 2, (s + 2) % NBUF)

    fire_gathers(0, 0)
    fire_gathers(1, 1)

    run_chunk(0, 0, head=True, tail=False, unroll=24)
    run_chunk(1, 1, head=True, tail=False, unroll=24)
    run_chunk(2, 2, head=False, tail=False, unroll=24)
    run_chunk(3, 3, head=False, tail=False, unroll=24)

    def group(g, carry):
        for u in range(NBUF):
            run_chunk(g * NBUF + u, u, head=False, tail=False, unroll=NSLICE)
        return carry

    lax.fori_loop(1, NCHUNK // NBUF - 1, group, 0)

    run_chunk(NCHUNK - 4, 0, head=False, tail=False, unroll=24)
    run_chunk(NCHUNK - 3, 1, head=False, tail=False, unroll=24)
    run_chunk(NCHUNK - 2, 2, head=False, tail=True, unroll=24)
    run_chunk(NCHUNK - 1, 3, head=False, tail=True, unroll=24)

    wait_scatter(NCHUNK - 2, 2)
    wait_scatter(NCHUNK - 1, 3)


@jax.jit
def kernel(x, table):
    pe = jnp.asarray(_PE)
    xt = x.reshape(BATCH, VOCAB // CHUNK, CHUNK).transpose(1, 0, 2).reshape(-1)
    mesh = plsc.VectorSubcoreMesh(core_axis_name="c", subcore_axis_name="s")
    f = pl.kernel(
        _body,
        out_type=jax.ShapeDtypeStruct((BATCH * VOCAB, D_MODEL), jnp.float32),
        mesh=mesh,
        scratch_types=[
            pltpu.VMEM((NCHUNK * ROWS,), jnp.int32),
            pltpu.VMEM((NCHUNK, ROWS), jnp.int32),
            pltpu.VMEM((NBUF, CHUNK, D_MODEL), jnp.float32),
            pltpu.VMEM((NBUF, ROWS, D_MODEL), jnp.float32),
            pltpu.SemaphoreType.DMA((NBUF,)),
            pltpu.SemaphoreType.DMA((NBUF,)),
        ],
    )
    return f(xt, table, pe).reshape(BATCH, VOCAB, D_MODEL)


# speedup vs baseline: 1.0106x; 1.0106x over previous
import numpy as np
import jax
import jax.numpy as jnp
from jax import lax
from jax.experimental import pallas as pl
from jax.experimental.pallas import tpu as pltpu
from jax.experimental.pallas import tpu_sc as plsc

VOCAB = 8192
D_MODEL = 768
BATCH = 4

NUM_CORES = 2
NUM_SUBCORES = 16
LANES = 16
NW = NUM_CORES * NUM_SUBCORES
POS_PER_W = VOCAB // NW
CHUNK = 8
NCHUNK = POS_PER_W // CHUNK
NBUF = 4
ROWS = BATCH * CHUNK
NSLICE = D_MODEL // LANES


def _make_pe() -> np.ndarray:
    pos = np.arange(VOCAB, dtype=np.float32)[:, None]
    i = np.arange(D_MODEL, dtype=np.float32)[None, :]
    denom = np.power(np.float32(10000.0),
                     (np.float32(2.0) * i) / np.float32(D_MODEL))
    angle = pos / denom
    even = (np.arange(D_MODEL)[None, :] % 2) == 0
    return np.where(even, np.sin(angle), np.cos(angle)).astype(np.float32)


_PE = _make_pe()


def _body(xt_hbm, table_hbm, pe_hbm, out_hbm,
          idx_f, pe_v, rows_v, gsem, ssem):
    w = lax.axis_index("s") * NUM_CORES + lax.axis_index("c")
    base = w * POS_PER_W

    pltpu.sync_copy(xt_hbm.at[pl.ds(w * NCHUNK * ROWS, NCHUNK * ROWS)], idx_f)


    def fire_gathers(c, s):
        pos = base + c * CHUNK
        pltpu.async_copy(pe_hbm.at[pl.ds(pos, CHUNK)], pe_v.at[s], gsem.at[s])
        pltpu.async_copy(table_hbm.at[idx_f.at[pl.ds(c * ROWS, ROWS)]],
                         rows_v.at[s], gsem.at[s])

    def wait_gathers(c, s):
        pos = base + c * CHUNK
        pltpu.make_async_copy(
            pe_hbm.at[pl.ds(pos, CHUNK)], pe_v.at[s], gsem.at[s]).wait()
        pltpu.make_async_copy(
            table_hbm.at[idx_f.at[pl.ds(c * ROWS, ROWS)]],
            rows_v.at[s], gsem.at[s]).wait()

    def fire_scatter(c, s):
        pos = base + c * CHUNK
        for b in range(BATCH):
            pltpu.async_copy(rows_v.at[s, pl.ds(b * CHUNK, CHUNK)],
                             out_hbm.at[b, pl.ds(pos, CHUNK)], ssem.at[s])

    def wait_scatter(c, s):
        pos = base + c * CHUNK
        for b in range(BATCH):
            pltpu.make_async_copy(
                rows_v.at[s, pl.ds(b * CHUNK, CHUNK)],
                out_hbm.at[b, pl.ds(pos, CHUNK)], ssem.at[s]).wait()

    def add_chunk(s, unroll):
        def jbody(j, carry):
            def kbody(k, carry2):
                for u in range(unroll):
                    sl = pl.ds(k * (unroll * LANES) + u * LANES, LANES)
                    p = pe_v[s, j, sl]
                    for b in range(BATCH):
                        rows_v[s, b * CHUNK + j, sl] = \
                            rows_v[s, b * CHUNK + j, sl] + p
                return carry2
            return lax.fori_loop(0, NSLICE // unroll, kbody, carry)
        lax.fori_loop(0, CHUNK, jbody, 0)

    def run_chunk(c, s, head, tail, unroll):
        wait_gathers(c, s)
        add_chunk(s, unroll)
        fire_scatter(c, s)
        if not head:
            wait_scatter(c - 2, (s + 2) % NBUF)
        if not tail:
            fire_gathers(c + 2, (s + 2) % NBUF)

    fire_gathers(0, 0)
    fire_gathers(1, 1)

    run_chunk(0, 0, head=True, tail=False, unroll=24)
    run_chunk(1, 1, head=True, tail=False, unroll=24)
    run_chunk(2, 2, head=False, tail=False, unroll=24)
    run_chunk(3, 3, head=False, tail=False, unroll=24)

    def group(g, carry):
        for u in range(NBUF):
            run_chunk(g * NBUF + u, u, head=False, tail=False, unroll=NSLICE)
        return carry

    lax.fori_loop(1, NCHUNK // NBUF - 1, group, 0)

    run_chunk(NCHUNK - 4, 0, head=False, tail=False, unroll=24)
    run_chunk(NCHUNK - 3, 1, head=False, tail=False, unroll=24)
    run_chunk(NCHUNK - 2, 2, head=False, tail=True, unroll=24)
    run_chunk(NCHUNK - 1, 3, head=False, tail=True, unroll=24)

    wait_scatter(NCHUNK - 2, 2)
    wait_scatter(NCHUNK - 1, 3)


@jax.jit
def kernel(x, table):
    pe = jnp.asarray(_PE)
    xt = x.reshape(BATCH, VOCAB // CHUNK, CHUNK).transpose(1, 0, 2).reshape(-1)
    mesh = plsc.VectorSubcoreMesh(core_axis_name="c", subcore_axis_name="s")
    f = pl.kernel(
        _body,
        out_type=jax.ShapeDtypeStruct((BATCH, VOCAB, D_MODEL), jnp.float32),
        mesh=mesh,
        scratch_types=[
            pltpu.VMEM((NCHUNK * ROWS,), jnp.int32),
            pltpu.VMEM((NBUF, CHUNK, D_MODEL), jnp.float32),
            pltpu.VMEM((NBUF, ROWS, D_MODEL), jnp.float32),
            pltpu.SemaphoreType.DMA((NBUF,)),
            pltpu.SemaphoreType.DMA((NBUF,)),
        ],
    )
    return f(xt, table, pe)


# speedup vs baseline: 1.0552x; 1.0441x over previous
import numpy as np
import jax
import jax.numpy as jnp
from jax import lax
from jax.experimental import pallas as pl
from jax.experimental.pallas import tpu as pltpu
from jax.experimental.pallas import tpu_sc as plsc

VOCAB = 8192
D_MODEL = 768
BATCH = 4

NUM_CORES = 2
NUM_SUBCORES = 16
LANES = 16
NW = NUM_CORES * NUM_SUBCORES
POS_PER_W = VOCAB // NW
CHUNK = 8
NCHUNK = POS_PER_W // CHUNK
NBUF = 4
NSLICE = D_MODEL // LANES


def _make_pe() -> np.ndarray:
    pos = np.arange(VOCAB, dtype=np.float32)[:, None]
    i = np.arange(D_MODEL, dtype=np.float32)[None, :]
    denom = np.power(np.float32(10000.0),
                     (np.float32(2.0) * i) / np.float32(D_MODEL))
    angle = pos / denom
    even = (np.arange(D_MODEL)[None, :] % 2) == 0
    return np.where(even, np.sin(angle), np.cos(angle)).astype(np.float32)


_PE = _make_pe()


def _body(x_hbm, table_hbm, pe_hbm, out_hbm, idx_v, pe_v, rows_v, gsem, ssem):
    w = lax.axis_index("s") * NUM_CORES + lax.axis_index("c")
    base = w * POS_PER_W

    for b in range(BATCH):
        pltpu.sync_copy(x_hbm.at[b, pl.ds(base, POS_PER_W)], idx_v.at[b])

    def fire_gathers(c, s):
        pos = base + c * CHUNK
        pltpu.async_copy(pe_hbm.at[pl.ds(pos, CHUNK)], pe_v.at[s], gsem.at[s])
        for b in range(BATCH):
            pltpu.async_copy(
                table_hbm.at[idx_v.at[b, pl.ds(c * CHUNK, CHUNK)]],
                rows_v.at[s, b], gsem.at[s])

    def wait_gathers(c, s):
        pos = base + c * CHUNK
        pltpu.make_async_copy(
            pe_hbm.at[pl.ds(pos, CHUNK)], pe_v.at[s], gsem.at[s]).wait()
        for b in range(BATCH):
            pltpu.make_async_copy(
                table_hbm.at[idx_v.at[b, pl.ds(c * CHUNK, CHUNK)]],
                rows_v.at[s, b], gsem.at[s]).wait()

    def fire_scatters(c, s):
        pos = base + c * CHUNK
        for b in range(BATCH):
            pltpu.async_copy(rows_v.at[s, b],
                             out_hbm.at[b, pl.ds(pos, CHUNK)], ssem.at[s])

    def wait_scatters(c, s):
        pos = base + c * CHUNK
        for b in range(BATCH):
            pltpu.make_async_copy(
                rows_v.at[s, b],
                out_hbm.at[b, pl.ds(pos, CHUNK)], ssem.at[s]).wait()

    def add_chunk(s, unroll):
        def jbody(j, carry):
            def kbody(k, carry2):
                for u in range(unroll):
                    sl = pl.ds(k * (unroll * LANES) + u * LANES, LANES)
                    p = pe_v[s, j, sl]
                    for b in range(BATCH):
                        rows_v[s, b, j, sl] = rows_v[s, b, j, sl] + p
                return carry2
            return lax.fori_loop(0, NSLICE // unroll, kbody, carry)
        lax.fori_loop(0, CHUNK, jbody, 0)

    def run_chunk(c, s, head, tail, unroll):
        wait_gathers(c, s)
        add_chunk(s, unroll)
        fire_scatters(c, s)
        if not head:
            wait_scatters(c - 2, (s + 2) % NBUF)
        if not tail:
            fire_gathers(c + 2, (s + 2) % NBUF)

    fire_gathers(0, 0)
    fire_gathers(1, 1)

    run_chunk(0, 0, head=True, tail=False, unroll=24)
    run_chunk(1, 1, head=True, tail=False, unroll=24)
    run_chunk(2, 2, head=False, tail=False, unroll=NSLICE)
    run_chunk(3, 3, head=False, tail=False, unroll=NSLICE)

    def group(g, carry):
        for u in range(NBUF):
            run_chunk(g * NBUF + u, u, head=False, tail=False, unroll=NSLICE)
        return carry

    lax.fori_loop(1, NCHUNK // NBUF - 1, group, 0)

    run_chunk(NCHUNK - 4, 0, head=False, tail=False, unroll=NSLICE)
    run_chunk(NCHUNK - 3, 1, head=False, tail=False, unroll=NSLICE)
    run_chunk(NCHUNK - 2, 2, head=False, tail=True, unroll=24)
    run_chunk(NCHUNK - 1, 3, head=False, tail=True, unroll=24)

    wait_scatters(NCHUNK - 2, 2)
    wait_scatters(NCHUNK - 1, 3)


@jax.jit
def kernel(x, table):
    pe = jnp.asarray(_PE)
    mesh = plsc.VectorSubcoreMesh(core_axis_name="c", subcore_axis_name="s")
    f = pl.kernel(
        _body,
        out_type=jax.ShapeDtypeStruct((BATCH, VOCAB, D_MODEL), jnp.float32),
        mesh=mesh,
        scratch_types=[
            pltpu.VMEM((BATCH, POS_PER_W), jnp.int32),
            pltpu.VMEM((NBUF, CHUNK, D_MODEL), jnp.float32),
            pltpu.VMEM((NBUF, BATCH, CHUNK, D_MODEL), jnp.float32),
            pltpu.SemaphoreType.DMA((NBUF,)),
            pltpu.SemaphoreType.DMA((NBUF,)),
        ],
    )
    return f(x, table, pe)


# speedup vs baseline: 1.0922x; 1.0351x over previous
import numpy as np
import jax
import jax.numpy as jnp
from jax import lax
from jax.experimental import pallas as pl
from jax.experimental.pallas import tpu as pltpu
from jax.experimental.pallas import tpu_sc as plsc

VOCAB = 8192
D_MODEL = 768
BATCH = 4

NUM_CORES = 2
NUM_SUBCORES = 16
LANES = 16
NW = NUM_CORES * NUM_SUBCORES
POS_PER_W = VOCAB // NW
CHUNK = 8
NCHUNK = POS_PER_W // CHUNK
NBUF = 4
NSLICE = D_MODEL // LANES


def _make_pe() -> np.ndarray:
    pos = np.arange(VOCAB, dtype=np.float32)[:, None]
    i = np.arange(D_MODEL, dtype=np.float32)[None, :]
    denom = np.power(np.float32(10000.0),
                     (np.float32(2.0) * i) / np.float32(D_MODEL))
    angle = pos / denom
    even = (np.arange(D_MODEL)[None, :] % 2) == 0
    return np.where(even, np.sin(angle), np.cos(angle)).astype(np.float32)


_PE = _make_pe()


def _body(x_hbm, table_hbm, pe_hbm, out_hbm, idx_v, pe_v, rows_v, gsem, ssem):
    w = lax.axis_index("s") * NUM_CORES + lax.axis_index("c")
    base = w * POS_PER_W

    for b in range(BATCH):
        pltpu.sync_copy(x_hbm.at[b, pl.ds(base, POS_PER_W)], idx_v.at[b])

    def fire_gathers(c, s):
        pos = base + c * CHUNK
        pltpu.async_copy(pe_hbm.at[pl.ds(pos, CHUNK)], pe_v.at[s], gsem.at[s])
        for b in range(BATCH):
            pltpu.async_copy(
                table_hbm.at[idx_v.at[b, pl.ds(c * CHUNK, CHUNK)]],
                rows_v.at[s, b], gsem.at[s])

    def wait_gathers(c, s):
        pos = base + c * CHUNK
        pltpu.make_async_copy(
            pe_hbm.at[pl.ds(pos, CHUNK)], pe_v.at[s], gsem.at[s]).wait()
        for b in range(BATCH):
            pltpu.make_async_copy(
                table_hbm.at[idx_v.at[b, pl.ds(c * CHUNK, CHUNK)]],
                rows_v.at[s, b], gsem.at[s]).wait()

    def fire_scatters(c, s):
        pos = base + c * CHUNK
        for b in range(BATCH):
            pltpu.async_copy(rows_v.at[s, b],
                             out_hbm.at[b, pl.ds(pos, CHUNK)], ssem.at[s])

    def wait_scatters(c, s):
        pos = base + c * CHUNK
        for b in range(BATCH):
            pltpu.make_async_copy(
                rows_v.at[s, b],
                out_hbm.at[b, pl.ds(pos, CHUNK)], ssem.at[s]).wait()

    def add_chunk(s, unroll):
        def jbody(j, carry):
            def kbody(k, carry2):
                for u in range(unroll):
                    sl = pl.ds(k * (unroll * LANES) + u * LANES, LANES)
                    p = pe_v[s, j, sl]
                    for b in range(BATCH):
                        rows_v[s, b, j, sl] = rows_v[s, b, j, sl] + p
                return carry2
            return lax.fori_loop(0, NSLICE // unroll, kbody, carry)
        lax.fori_loop(0, CHUNK, jbody, 0)

    def run_chunk(c, s, head, tail, unroll):
        wait_gathers(c, s)
        add_chunk(s, unroll)
        fire_scatters(c, s)
        if not head:
            wait_scatters(c - 2, (s + 2) % NBUF)
        if not tail:
            fire_gathers(c + 2, (s + 2) % NBUF)

    fire_gathers(0, 0)
    fire_gathers(1, 1)

    run_chunk(0, 0, head=True, tail=False, unroll=NSLICE)
    run_chunk(1, 1, head=True, tail=False, unroll=NSLICE)
    run_chunk(2, 2, head=False, tail=False, unroll=NSLICE)
    run_chunk(3, 3, head=False, tail=False, unroll=NSLICE)

    def group(g, carry):
        for u in range(NBUF):
            run_chunk(g * NBUF + u, u, head=False, tail=False, unroll=NSLICE)
        return carry

    lax.fori_loop(1, NCHUNK // NBUF - 1, group, 0)

    run_chunk(NCHUNK - 4, 0, head=False, tail=False, unroll=NSLICE)
    run_chunk(NCHUNK - 3, 1, head=False, tail=False, unroll=NSLICE)
    run_chunk(NCHUNK - 2, 2, head=False, tail=True, unroll=NSLICE)
    run_chunk(NCHUNK - 1, 3, head=False, tail=True, unroll=NSLICE)

    wait_scatters(NCHUNK - 2, 2)
    wait_scatters(NCHUNK - 1, 3)


@jax.jit
def kernel(x, table):
    pe = jnp.asarray(_PE)
    mesh = plsc.VectorSubcoreMesh(core_axis_name="c", subcore_axis_name="s")
    f = pl.kernel(
        _body,
        out_type=jax.ShapeDtypeStruct((BATCH, VOCAB, D_MODEL), jnp.float32),
        mesh=mesh,
        scratch_types=[
            pltpu.VMEM((BATCH, POS_PER_W), jnp.int32),
            pltpu.VMEM((NBUF, CHUNK, D_MODEL), jnp.float32),
            pltpu.VMEM((NBUF, BATCH, CHUNK, D_MODEL), jnp.float32),
            pltpu.SemaphoreType.DMA((NBUF,)),
            pltpu.SemaphoreType.DMA((NBUF,)),
        ],
    )
    return f(x, table, pe)


# speedup vs baseline: 1.0967x; 1.0041x over previous
import numpy as np
import jax
import jax.numpy as jnp
from jax import lax
from jax.experimental import pallas as pl
from jax.experimental.pallas import tpu as pltpu
from jax.experimental.pallas import tpu_sc as plsc

VOCAB = 8192
D_MODEL = 768
BATCH = 4

NUM_CORES = 2
NUM_SUBCORES = 16
LANES = 16
NW = NUM_CORES * NUM_SUBCORES
POS_PER_W = VOCAB // NW
CHUNK = 8
NCHUNK = POS_PER_W // CHUNK
NBUF = 4
NSLICE = D_MODEL // LANES


def _make_pe() -> np.ndarray:
    pos = np.arange(VOCAB, dtype=np.float32)[:, None]
    i = np.arange(D_MODEL, dtype=np.float32)[None, :]
    denom = np.power(np.float32(10000.0),
                     (np.float32(2.0) * i) / np.float32(D_MODEL))
    angle = pos / denom
    even = (np.arange(D_MODEL)[None, :] % 2) == 0
    return np.where(even, np.sin(angle), np.cos(angle)).astype(np.float32)


_PE = _make_pe()


def _body(x_hbm, table_hbm, pe_hbm, out_hbm, idx_v, pe_v, rows_v, gsem, ssem):
    w = lax.axis_index("s") * NUM_CORES + lax.axis_index("c")
    base = w * POS_PER_W

    for b in range(BATCH):
        pltpu.sync_copy(x_hbm.at[b, pl.ds(base, POS_PER_W)], idx_v.at[b])

    def fire_gathers(c, s):
        pos = base + c * CHUNK
        pltpu.async_copy(pe_hbm.at[pl.ds(pos, CHUNK)], pe_v.at[s], gsem.at[s])
        for b in range(BATCH):
            pltpu.async_copy(
                table_hbm.at[idx_v.at[b, pl.ds(c * CHUNK, CHUNK)]],
                rows_v.at[s, b], gsem.at[s])

    def wait_gathers(c, s):
        pos = base + c * CHUNK
        pltpu.make_async_copy(
            pe_hbm.at[pl.ds(pos, CHUNK)], pe_v.at[s], gsem.at[s]).wait()
        pltpu.make_async_copy(
            out_hbm.at[:, pl.ds(0, CHUNK)], rows_v.at[s], gsem.at[s]).wait()

    def fire_scatters(c, s):
        pos = base + c * CHUNK
        for b in range(BATCH):
            pltpu.async_copy(rows_v.at[s, b],
                             out_hbm.at[b, pl.ds(pos, CHUNK)], ssem.at[s])

    def wait_scatters(c, s):
        pos = base + c * CHUNK
        pltpu.make_async_copy(
            rows_v.at[s], out_hbm.at[:, pl.ds(pos, CHUNK)], ssem.at[s]).wait()

    def add_chunk(s, unroll):
        def jbody(j, carry):
            def kbody(k, carry2):
                for u in range(unroll):
                    sl = pl.ds(k * (unroll * LANES) + u * LANES, LANES)
                    p = pe_v[s, j, sl]
                    for b in range(BATCH):
                        rows_v[s, b, j, sl] = rows_v[s, b, j, sl] + p
                return carry2
            return lax.fori_loop(0, NSLICE // unroll, kbody, carry)
        lax.fori_loop(0, CHUNK, jbody, 0)

    def run_chunk(c, s, head, tail, unroll):
        wait_gathers(c, s)
        add_chunk(s, unroll)
        fire_scatters(c, s)
        if not head:
            wait_scatters(c - 2, (s + 2) % NBUF)
        if not tail:
            fire_gathers(c + 2, (s + 2) % NBUF)

    fire_gathers(0, 0)
    fire_gathers(1, 1)

    run_chunk(0, 0, head=True, tail=False, unroll=NSLICE)
    run_chunk(1, 1, head=True, tail=False, unroll=NSLICE)
    run_chunk(2, 2, head=False, tail=False, unroll=NSLICE)
    run_chunk(3, 3, head=False, tail=False, unroll=NSLICE)

    def group(g, carry):
        for u in range(NBUF):
            run_chunk(g * NBUF + u, u, head=False, tail=False, unroll=NSLICE)
        return carry

    lax.fori_loop(1, NCHUNK // NBUF - 1, group, 0)

    run_chunk(NCHUNK - 4, 0, head=False, tail=False, unroll=NSLICE)
    run_chunk(NCHUNK - 3, 1, head=False, tail=False, unroll=NSLICE)
    run_chunk(NCHUNK - 2, 2, head=False, tail=True, unroll=NSLICE)
    run_chunk(NCHUNK - 1, 3, head=False, tail=True, unroll=NSLICE)

    wait_scatters(NCHUNK - 2, 2)
    wait_scatters(NCHUNK - 1, 3)


@jax.jit
def kernel(x, table):
    pe = jnp.asarray(_PE)
    mesh = plsc.VectorSubcoreMesh(core_axis_name="c", subcore_axis_name="s")
    f = pl.kernel(
        _body,
        out_type=jax.ShapeDtypeStruct((BATCH, VOCAB, D_MODEL), jnp.float32),
        mesh=mesh,
        scratch_types=[
            pltpu.VMEM((BATCH, POS_PER_W), jnp.int32),
            pltpu.VMEM((NBUF, CHUNK, D_MODEL), jnp.float32),
            pltpu.VMEM((NBUF, BATCH, CHUNK, D_MODEL), jnp.float32),
            pltpu.SemaphoreType.DMA((NBUF,)),
            pltpu.SemaphoreType.DMA((NBUF,)),
        ],
    )
    return f(x, table, pe)


# speedup vs baseline: 1.1107x; 1.0127x over previous
import numpy as np
import jax
import jax.numpy as jnp
from jax import lax
from jax.experimental import pallas as pl
from jax.experimental.pallas import tpu as pltpu
from jax.experimental.pallas import tpu_sc as plsc

VOCAB = 8192
D_MODEL = 768
BATCH = 4

NUM_CORES = 2
NUM_SUBCORES = 16
LANES = 16
NW = NUM_CORES * NUM_SUBCORES
POS_PER_W = VOCAB // NW
CHUNK = 8
NCHUNK = POS_PER_W // CHUNK
NBUF = 4
NSLICE = D_MODEL // LANES


def _make_pe() -> np.ndarray:
    pos = np.arange(VOCAB, dtype=np.float32)[:, None]
    i = np.arange(D_MODEL, dtype=np.float32)[None, :]
    denom = np.power(np.float32(10000.0),
                     (np.float32(2.0) * i) / np.float32(D_MODEL))
    angle = pos / denom
    even = (np.arange(D_MODEL)[None, :] % 2) == 0
    return np.where(even, np.sin(angle), np.cos(angle)).astype(np.float32)


_PE = _make_pe()


def _body(x_hbm, table_hbm, pe_hbm, out_hbm, idx_v, pe_v, rows_v, gsem, ssem):
    w = lax.axis_index("s") * NUM_CORES + lax.axis_index("c")
    base = w * POS_PER_W

    for b in range(BATCH):
        pltpu.sync_copy(x_hbm.at[b, pl.ds(base, POS_PER_W)], idx_v.at[b])

    def fire_gathers(c, s):
        pos = base + c * CHUNK
        pltpu.async_copy(pe_hbm.at[pl.ds(pos, CHUNK)], pe_v.at[s], gsem.at[s])
        for b in range(BATCH):
            pltpu.async_copy(
                table_hbm.at[idx_v.at[b, pl.ds(c * CHUNK, CHUNK)]],
                rows_v.at[s, b], gsem.at[s])

    def wait_gathers(c, s):
        pos = base + c * CHUNK
        pltpu.make_async_copy(
            pe_hbm.at[pl.ds(pos, CHUNK)], pe_v.at[s], gsem.at[s]).wait()
        pltpu.make_async_copy(
            out_hbm.at[:, pl.ds(0, CHUNK)], rows_v.at[s], gsem.at[s]).wait()

    def fire_scatters(c, s):
        pos = base + c * CHUNK
        for b in range(BATCH):
            pltpu.async_copy(rows_v.at[s, b],
                             out_hbm.at[b, pl.ds(pos, CHUNK)], ssem.at[s])

    def wait_scatters(c, s):
        pos = base + c * CHUNK
        pltpu.make_async_copy(
            rows_v.at[s], out_hbm.at[:, pl.ds(pos, CHUNK)], ssem.at[s]).wait()

    def add_chunk(s, unroll):
        def jbody(j, carry):
            def kbody(k, carry2):
                for u in range(unroll):
                    sl = pl.ds(k * (unroll * LANES) + u * LANES, LANES)
                    p = pe_v[s, j, sl]
                    for b in range(BATCH):
                        rows_v[s, b, j, sl] = rows_v[s, b, j, sl] + p
                return carry2
            return lax.fori_loop(0, NSLICE // unroll, kbody, carry)
        lax.fori_loop(0, CHUNK, jbody, 0)

    def run_chunk(c, s, head, tail, unroll):
        wait_gathers(c, s)
        if not head:
            wait_scatters(c - 2, (s + 2) % NBUF)
        if not tail:
            fire_gathers(c + 2, (s + 2) % NBUF)
        add_chunk(s, unroll)
        fire_scatters(c, s)

    fire_gathers(0, 0)
    fire_gathers(1, 1)

    run_chunk(0, 0, head=True, tail=False, unroll=NSLICE)
    run_chunk(1, 1, head=True, tail=False, unroll=NSLICE)
    run_chunk(2, 2, head=False, tail=False, unroll=NSLICE)
    run_chunk(3, 3, head=False, tail=False, unroll=NSLICE)

    def group(g, carry):
        for u in range(NBUF):
            run_chunk(g * NBUF + u, u, head=False, tail=False, unroll=NSLICE)
        return carry

    lax.fori_loop(1, NCHUNK // NBUF - 1, group, 0)

    run_chunk(NCHUNK - 4, 0, head=False, tail=False, unroll=NSLICE)
    run_chunk(NCHUNK - 3, 1, head=False, tail=False, unroll=NSLICE)
    run_chunk(NCHUNK - 2, 2, head=False, tail=True, unroll=NSLICE)
    run_chunk(NCHUNK - 1, 3, head=False, tail=True, unroll=NSLICE)

    wait_scatters(NCHUNK - 2, 2)
    wait_scatters(NCHUNK - 1, 3)


@jax.jit
def kernel(x, table):
    pe = jnp.asarray(_PE)
    mesh = plsc.VectorSubcoreMesh(core_axis_name="c", subcore_axis_name="s")
    f = pl.kernel(
        _body,
        out_type=jax.ShapeDtypeStruct((BATCH, VOCAB, D_MODEL), jnp.float32),
        mesh=mesh,
        scratch_types=[
            pltpu.VMEM((BATCH, POS_PER_W), jnp.int32),
            pltpu.VMEM((NBUF, CHUNK, D_MODEL), jnp.float32),
            pltpu.VMEM((NBUF, BATCH, CHUNK, D_MODEL), jnp.float32),
            pltpu.SemaphoreType.DMA((NBUF,)),
            pltpu.SemaphoreType.DMA((NBUF,)),
        ],
    )
    return f(x, table, pe)
